# Initial kernel scaffold; baseline (speedup 1.0000x reference)
import jax
import jax.numpy as jnp
from jax.experimental import pallas as pl


def kernel(x, embedding):
    raise NotImplementedError("write your pallas kernel here")


# speedup vs baseline: 1.5479x; 1.5479x over previous
import functools

import jax
import jax.numpy as jnp
from jax import lax
from jax.experimental import pallas as pl
from jax.experimental.pallas import tpu as pltpu
from jax.experimental.pallas import tpu_sc as plsc

_SC_INFO = plsc.get_sparse_core_info()
_NUM_CORES = _SC_INFO.num_cores
_NUM_SUBCORES = _SC_INFO.num_subcores
_NUM_WORKERS = _NUM_CORES * _NUM_SUBCORES


@functools.lru_cache(maxsize=None)
def _make_broadcast(batch: int, length: int, features: int):
    assert length % _NUM_WORKERS == 0
    rows_per_worker = length // _NUM_WORKERS
    chunk = rows_per_worker
    while chunk * features * 4 * 2 > 256 * 1024:
        chunk //= 2
    assert rows_per_worker % chunk == 0
    nchunk = rows_per_worker // chunk

    mesh = plsc.VectorSubcoreMesh(core_axis_name="c", subcore_axis_name="s")

    @functools.partial(
        pl.kernel,
        mesh=mesh,
        out_type=jax.ShapeDtypeStruct((batch, length, features), jnp.float32),
        scratch_types=[
            pltpu.VMEM((chunk, features), jnp.float32),
            pltpu.VMEM((chunk, features), jnp.float32),
            pltpu.SemaphoreType.DMA,
            pltpu.SemaphoreType.DMA,
        ],
    )
    def broadcast_rows(emb_hbm, out_hbm, buf0, buf1, sem_r, sem_w):
        del buf1, sem_r, sem_w
        wid = lax.axis_index("s") * _NUM_CORES + lax.axis_index("c")
        base = wid * rows_per_worker
        for g in range(nchunk):
            row = base + g * chunk
            pltpu.sync_copy(emb_hbm.at[pl.ds(row, chunk)], buf0)
            for b in range(batch):
                pltpu.sync_copy(buf0, out_hbm.at[b, pl.ds(row, chunk)])

    return broadcast_rows


def kernel(x, embedding):
    batch, length = x.shape[0], x.shape[1]
    features = embedding.shape[1]
    return _make_broadcast(batch, length, features)(embedding)


# speedup vs baseline: 1.5642x; 1.0105x over previous
import functools

import jax
import jax.numpy as jnp
from jax import lax
from jax.experimental import pallas as pl
from jax.experimental.pallas import tpu as pltpu
from jax.experimental.pallas import tpu_sc as plsc

_SC_INFO = plsc.get_sparse_core_info()
_NUM_CORES = _SC_INFO.num_cores
_NUM_SUBCORES = _SC_INFO.num_subcores
_NUM_WORKERS = _NUM_CORES * _NUM_SUBCORES


@functools.lru_cache(maxsize=None)
def _make_broadcast(batch: int, length: int, features: int):
    assert length % _NUM_WORKERS == 0
    rows_per_worker = length // _NUM_WORKERS
    chunk = rows_per_worker
    while chunk * features * 4 * 2 > 256 * 1024:
        chunk //= 2
    assert rows_per_worker % chunk == 0
    nchunk = rows_per_worker // chunk

    mesh = plsc.VectorSubcoreMesh(core_axis_name="c", subcore_axis_name="s")

    @functools.partial(
        pl.kernel,
        mesh=mesh,
        out_type=jax.ShapeDtypeStruct((batch, length, features), jnp.float32),
        scratch_types=[
            pltpu.VMEM((chunk, features), jnp.float32),
            pltpu.VMEM((chunk, features), jnp.float32),
            pltpu.SemaphoreType.DMA,
            pltpu.SemaphoreType.DMA,
        ],
    )
    def broadcast_rows(emb_hbm, out_hbm, buf0, buf1, sem_r, sem_w):
        wid = lax.axis_index("s") * _NUM_CORES + lax.axis_index("c")
        base = wid * rows_per_worker
        bufs = (buf0, buf1)
        sems = (sem_r, sem_w)
        pending = [[], []]
        for g in range(nchunk):
            row = base + g * chunk
            slot = g % 2
            for d in pending[slot]:
                d.wait()
            pltpu.sync_copy(emb_hbm.at[pl.ds(row, chunk)], bufs[slot])
            pending[slot] = [
                pltpu.async_copy(
                    bufs[slot], out_hbm.at[b, pl.ds(row, chunk)], sems[slot])
                for b in range(batch)
            ]
        for p in pending:
            for d in p:
                d.wait()

    return broadcast_rows


def kernel(x, embedding):
    batch, length = x.shape[0], x.shape[1]
    features = embedding.shape[1]
    return _make_broadcast(batch, length, features)(embedding)


# speedup vs baseline: 1.5950x; 1.0197x over previous
import functools

import jax
import jax.numpy as jnp
from jax import lax
from jax.experimental import pallas as pl
from jax.experimental.pallas import tpu as pltpu
from jax.experimental.pallas import tpu_sc as plsc

_SC_INFO = plsc.get_sparse_core_info()
_NUM_CORES = _SC_INFO.num_cores
_NUM_SUBCORES = _SC_INFO.num_subcores
_NUM_WORKERS = _NUM_CORES * _NUM_SUBCORES


@functools.lru_cache(maxsize=None)
def _make_broadcast(batch: int, length: int, features: int):
    assert length % _NUM_WORKERS == 0
    rows_per_worker = length // _NUM_WORKERS
    chunk = rows_per_worker
    while chunk * features * 4 * 2 > 256 * 1024:
        chunk //= 2
    assert rows_per_worker % chunk == 0
    nchunk = rows_per_worker // chunk

    mesh = plsc.VectorSubcoreMesh(core_axis_name="c", subcore_axis_name="s")

    @functools.partial(
        pl.kernel,
        mesh=mesh,
        out_type=jax.ShapeDtypeStruct((batch, length, features), jnp.float32),
        scratch_types=[
            pltpu.VMEM((chunk, features), jnp.float32),
            pltpu.VMEM((chunk, features), jnp.float32),
            pltpu.SemaphoreType.DMA,
            pltpu.SemaphoreType.DMA,
            pltpu.SemaphoreType.DMA,
        ],
    )
    def broadcast_rows(emb_hbm, out_hbm, buf0, buf1, sem_r, sem_w0, sem_w1):
        wid = lax.axis_index("s") * _NUM_CORES + lax.axis_index("c")
        base = wid * rows_per_worker
        bufs = (buf0, buf1)
        sems_w = (sem_w0, sem_w1)
        pending_w = [[], []]
        rd = pltpu.async_copy(emb_hbm.at[pl.ds(base, chunk)], buf0, sem_r)
        for g in range(nchunk):
            slot = g % 2
            rd.wait()
            if g + 1 < nchunk:
                nslot = (g + 1) % 2
                for d in pending_w[nslot]:
                    d.wait()
                rd = pltpu.async_copy(
                    emb_hbm.at[pl.ds(base + (g + 1) * chunk, chunk)],
                    bufs[nslot], sem_r)
            row = base + g * chunk
            pending_w[slot] = [
                pltpu.async_copy(
                    bufs[slot], out_hbm.at[b, pl.ds(row, chunk)], sems_w[slot])
                for b in range(batch)
            ]
        for p in pending_w:
            for d in p:
                d.wait()

    return broadcast_rows


def kernel(x, embedding):
    batch, length = x.shape[0], x.shape[1]
    features = embedding.shape[1]
    return _make_broadcast(batch, length, features)(embedding)
